# baseline (speedup 1.0000x reference)
import functools

import jax
import jax.numpy as jnp
from jax import lax
from jax.experimental import pallas as pl
from jax.experimental.pallas import tpu as pltpu
from jax.experimental.pallas import tpu_sc as plsc

NC = 2
NS = 16
NW = NC * NS

N_NODES = 10000
N_EDGES = 160000
D_FEAT = 128
EMB_DIM = 32
OUT_DIM = 32
NUM_RELS = 8

NBG = 320
GROUPS = 5
GB = N_EDGES // (NW * GROUPS)
ZCH = N_NODES // NS

@functools.cache
def _sc_kernels():
    mesh = plsc.VectorSubcoreMesh(core_axis_name="c", subcore_axis_name="s",
                                  num_cores=NC, num_subcores=NS)
    params = pltpu.CompilerParams(use_tc_tiling_on_sc=False)

    @functools.partial(
        pl.kernel,
        out_type=jax.ShapeDtypeStruct((N_NODES, EMB_DIM), jnp.float32),
        mesh=mesh,
        compiler_params=params,
        scratch_types=[
            pltpu.VMEM((NBG,), jnp.int32),
            pltpu.VMEM((NBG, EMB_DIM), jnp.float32),
            pltpu.SemaphoreType.DMA,
        ],
    )
    def _embed_gather(embed_hbm, idx_hbm, out_hbm, idx_v, rows_v, sem):
        w = lax.axis_index("c") * NS + lax.axis_index("s")
        start = jnp.minimum(w * NBG, N_NODES - NBG)
        pltpu.sync_copy(idx_hbm.at[pl.ds(start, NBG)], idx_v)
        pltpu.async_copy(embed_hbm.at[idx_v], rows_v, sem).wait()
        pltpu.sync_copy(rows_v, out_hbm.at[pl.ds(start, NBG)])

    @functools.partial(
        pl.kernel,
        out_type=jax.ShapeDtypeStruct((NC * N_NODES, OUT_DIM), jnp.float32),
        mesh=mesh,
        compiler_params=params,
        scratch_types=[
            pltpu.VMEM((GROUPS * GB,), jnp.int32),
            pltpu.VMEM((GROUPS * GB,), jnp.int32),
            pltpu.VMEM((GB, OUT_DIM), jnp.float32),
            pltpu.VMEM((GB, OUT_DIM), jnp.float32),
            pltpu.VMEM_SHARED((N_NODES, OUT_DIM), jnp.float32),
            pltpu.SemaphoreType.DMA,
            pltpu.SemaphoreType.DMA,
            pltpu.SemaphoreType.DMA,
            pltpu.SemaphoreType.DMA,
        ],
    )
    def _edge_agg(y_hbm, gi_hbm, dst_hbm, zeros_hbm, out_hbm,
                  gi_v, dst_v, r0, r1, acc_sh, sg0, sg1, ss0, ss1):
        c = lax.axis_index("c")
        s = lax.axis_index("s")
        w = c * NS + s
        ebase = w * (GROUPS * GB)

        pltpu.sync_copy(gi_hbm.at[pl.ds(ebase, GROUPS * GB)], gi_v)
        pltpu.sync_copy(dst_hbm.at[pl.ds(ebase, GROUPS * GB)], dst_v)

        bufs = [(r0, sg0, ss0), (r1, sg1, ss1)]
        gathers = {}
        for g in range(min(2, GROUPS)):
            rb, sg, _ = bufs[g % 2]
            gathers[g] = pltpu.async_copy(
                y_hbm.at[gi_v.at[pl.ds(g * GB, GB)]], rb, sg)

        pltpu.sync_copy(zeros_hbm.at[pl.ds(s * ZCH, ZCH)],
                        acc_sh.at[pl.ds(s * ZCH, ZCH)])
        plsc.subcore_barrier()

        scats = {}
        for g in range(GROUPS):
            rb, sg, ss = bufs[g % 2]
            gathers[g].wait()
            scats[g] = pltpu.async_copy(
                rb, acc_sh.at[dst_v.at[pl.ds(g * GB, GB)]], ss, add=True)
            if g + 2 < GROUPS:
                scats[g].wait()
                gathers[g + 2] = pltpu.async_copy(
                    y_hbm.at[gi_v.at[pl.ds((g + 2) * GB, GB)]], rb, sg)
        for g in range(max(0, GROUPS - 2), GROUPS):
            scats[g].wait()

        plsc.subcore_barrier()

        pltpu.sync_copy(acc_sh.at[pl.ds(s * ZCH, ZCH)],
                        out_hbm.at[pl.ds(c * N_NODES + s * ZCH, ZCH)])

    return _embed_gather, _edge_agg


def _dense_body(feat_ref, g_ref, t_ref, w_ref, wc_ref, sl_ref,
                y_ref, fc_ref):
    t1 = t_ref[:D_FEAT, :]
    t2 = t_ref[D_FEAT:, :]
    fea = (jnp.dot(feat_ref[...], t1, preferred_element_type=jnp.float32)
           + jnp.dot(g_ref[...], t2, preferred_element_type=jnp.float32))
    w8 = jnp.sum(wc_ref[...][:, :, None, None] * w_ref[...][None, :, :, :],
                 axis=1)
    wcat = jnp.concatenate([w8[r] for r in range(NUM_RELS)], axis=1)
    y_ref[...] = jnp.dot(fea, wcat, preferred_element_type=jnp.float32)
    curr = jnp.dot(fea, sl_ref[...], preferred_element_type=jnp.float32)
    fc_ref[...] = jnp.concatenate([fea, curr], axis=1)


_RB = 1000

_dense = pl.pallas_call(
    _dense_body,
    grid=(N_NODES // _RB,),
    in_specs=[
        pl.BlockSpec((_RB, D_FEAT), lambda i: (i, 0)),
        pl.BlockSpec((_RB, EMB_DIM), lambda i: (i, 0)),
        pl.BlockSpec((D_FEAT + EMB_DIM, OUT_DIM), lambda i: (0, 0)),
        pl.BlockSpec((4, OUT_DIM, OUT_DIM), lambda i: (0, 0, 0)),
        pl.BlockSpec((NUM_RELS, 4), lambda i: (0, 0)),
        pl.BlockSpec((OUT_DIM, OUT_DIM), lambda i: (0, 0)),
    ],
    out_specs=[
        pl.BlockSpec((_RB, NUM_RELS * OUT_DIM), lambda i: (i, 0)),
        pl.BlockSpec((_RB, 2 * OUT_DIM), lambda i: (i, 0)),
    ],
    out_shape=[
        jax.ShapeDtypeStruct((N_NODES, NUM_RELS * OUT_DIM), jnp.float32),
        jax.ShapeDtypeStruct((N_NODES, 2 * OUT_DIM), jnp.float32),
    ],
)


def _assemble_body(fc_ref, a0_ref, a1_ref, out_ref):
    out_ref[:, :OUT_DIM] = fc_ref[:, :OUT_DIM]
    out_ref[:, OUT_DIM:] = fc_ref[:, OUT_DIM:] + a0_ref[...] + a1_ref[...]


_AB = 2000
_ANB = N_NODES // _AB

_assemble = pl.pallas_call(
    _assemble_body,
    grid=(_ANB,),
    in_specs=[
        pl.BlockSpec((_AB, 2 * OUT_DIM), lambda i: (i, 0)),
        pl.BlockSpec((_AB, OUT_DIM), lambda i: (i, 0)),
        pl.BlockSpec((_AB, OUT_DIM), lambda i: (i + _ANB, 0)),
    ],
    out_specs=pl.BlockSpec((_AB, 2 * OUT_DIM), lambda i: (i, 0)),
    out_shape=jax.ShapeDtypeStruct((N_NODES, 2 * OUT_DIM), jnp.float32),
)


def kernel(feat, idx, edge_index, edge_type, embed, transform, weight,
           w_comp, self_loop_weight):
    src = edge_index[0]
    dst = edge_index[1]

    embed_gather, edge_agg = _sc_kernels()

    g = embed_gather(embed, idx.astype(jnp.int32))

    y, fc = _dense(feat, g, transform, weight, w_comp,
                   self_loop_weight)
    y_flat = y.reshape(NUM_RELS * N_NODES, OUT_DIM)

    gi = (src.astype(jnp.int32) * NUM_RELS
          + edge_type.astype(jnp.int32))
    dst_r = dst.astype(jnp.int32)
    zeros = jnp.zeros((N_NODES, OUT_DIM), jnp.float32)

    acc = edge_agg(y_flat, gi, dst_r, zeros)

    out = _assemble(fc, acc, acc)
    return out.reshape(-1, 2, OUT_DIM)


# speedup vs baseline: 1.0463x; 1.0463x over previous
import functools

import jax
import jax.numpy as jnp
from jax import lax
from jax.experimental import pallas as pl
from jax.experimental.pallas import tpu as pltpu
from jax.experimental.pallas import tpu_sc as plsc

NC = 2
NS = 16
NW = NC * NS

N_NODES = 10000
N_EDGES = 160000
D_FEAT = 128
EMB_DIM = 32
OUT_DIM = 32
NUM_RELS = 8

NBG = 320
GROUPS = 5
GB = N_EDGES // (NW * GROUPS)
ZCH = N_NODES // NS

@functools.cache
def _sc_kernels():
    mesh = plsc.VectorSubcoreMesh(core_axis_name="c", subcore_axis_name="s",
                                  num_cores=NC, num_subcores=NS)
    params = pltpu.CompilerParams(use_tc_tiling_on_sc=False)

    @functools.partial(
        pl.kernel,
        out_type=jax.ShapeDtypeStruct((N_NODES, EMB_DIM), jnp.float32),
        mesh=mesh,
        compiler_params=params,
        scratch_types=[
            pltpu.VMEM((NBG,), jnp.int32),
            pltpu.VMEM((NBG, EMB_DIM), jnp.float32),
            pltpu.SemaphoreType.DMA,
        ],
    )
    def _embed_gather(embed_hbm, idx_hbm, out_hbm, idx_v, rows_v, sem):
        w = lax.axis_index("c") * NS + lax.axis_index("s")
        start = jnp.minimum(w * NBG, N_NODES - NBG)
        pltpu.sync_copy(idx_hbm.at[pl.ds(start, NBG)], idx_v)
        pltpu.async_copy(embed_hbm.at[idx_v], rows_v, sem).wait()
        pltpu.sync_copy(rows_v, out_hbm.at[pl.ds(start, NBG)])

    @functools.partial(
        pl.kernel,
        out_type=jax.ShapeDtypeStruct((NC * N_NODES, OUT_DIM), jnp.float32),
        mesh=mesh,
        compiler_params=params,
        scratch_types=[
            pltpu.VMEM((GROUPS * GB,), jnp.int32),
            pltpu.VMEM((GROUPS * GB,), jnp.int32),
            pltpu.VMEM((GB, OUT_DIM), jnp.float32),
            pltpu.VMEM((GB, OUT_DIM), jnp.float32),
            pltpu.VMEM_SHARED((N_NODES, OUT_DIM), jnp.float32),
            pltpu.SemaphoreType.DMA,
            pltpu.SemaphoreType.DMA,
            pltpu.SemaphoreType.DMA,
            pltpu.SemaphoreType.DMA,
        ],
    )
    def _edge_agg(y_hbm, gi_hbm, dst_hbm, zeros_hbm, out_hbm,
                  gi_v, dst_v, r0, r1, acc_sh, sg0, sg1, ss0, ss1):
        c = lax.axis_index("c")
        s = lax.axis_index("s")
        w = c * NS + s
        ebase = w * (GROUPS * GB)

        pltpu.sync_copy(gi_hbm.at[pl.ds(ebase, GROUPS * GB)], gi_v)
        pltpu.sync_copy(dst_hbm.at[pl.ds(ebase, GROUPS * GB)], dst_v)

        bufs = [(r0, sg0, ss0), (r1, sg1, ss1)]
        gathers = {}
        for g in range(min(2, GROUPS)):
            rb, sg, _ = bufs[g % 2]
            gathers[g] = pltpu.async_copy(
                y_hbm.at[gi_v.at[pl.ds(g * GB, GB)]], rb, sg)

        pltpu.sync_copy(zeros_hbm.at[pl.ds(s * ZCH, ZCH)],
                        acc_sh.at[pl.ds(s * ZCH, ZCH)])
        plsc.subcore_barrier()

        scats = {}
        for g in range(GROUPS):
            rb, sg, ss = bufs[g % 2]
            gathers[g].wait()
            scats[g] = pltpu.async_copy(
                rb, acc_sh.at[dst_v.at[pl.ds(g * GB, GB)]], ss, add=True)
            if g + 2 < GROUPS:
                scats[g].wait()
                gathers[g + 2] = pltpu.async_copy(
                    y_hbm.at[gi_v.at[pl.ds((g + 2) * GB, GB)]], rb, sg)
        for g in range(max(0, GROUPS - 2), GROUPS):
            scats[g].wait()

        plsc.subcore_barrier()

        pltpu.sync_copy(acc_sh.at[pl.ds(s * ZCH, ZCH)],
                        out_hbm.at[pl.ds(c * N_NODES + s * ZCH, ZCH)])

    return _embed_gather, _edge_agg


def _dense_body(feat_ref, g_ref, t_ref, w_ref, wc_ref, sl_ref,
                y_ref, fc_ref):
    t1 = t_ref[:D_FEAT, :]
    t2 = t_ref[D_FEAT:, :]
    fea = (jnp.dot(feat_ref[...], t1, preferred_element_type=jnp.float32)
           + jnp.dot(g_ref[...], t2, preferred_element_type=jnp.float32))
    w8 = jnp.sum(wc_ref[...][:, :, None, None] * w_ref[...][None, :, :, :],
                 axis=1)
    wcat = jnp.concatenate([w8[r] for r in range(NUM_RELS)], axis=1)
    y_ref[...] = jnp.dot(fea, wcat, preferred_element_type=jnp.float32)
    curr = jnp.dot(fea, sl_ref[...], preferred_element_type=jnp.float32)
    fc_ref[...] = jnp.concatenate([fea, curr], axis=1)


_RB = 2000

_dense = pl.pallas_call(
    _dense_body,
    grid=(N_NODES // _RB,),
    in_specs=[
        pl.BlockSpec((_RB, D_FEAT), lambda i: (i, 0)),
        pl.BlockSpec((_RB, EMB_DIM), lambda i: (i, 0)),
        pl.BlockSpec((D_FEAT + EMB_DIM, OUT_DIM), lambda i: (0, 0)),
        pl.BlockSpec((4, OUT_DIM, OUT_DIM), lambda i: (0, 0, 0)),
        pl.BlockSpec((NUM_RELS, 4), lambda i: (0, 0)),
        pl.BlockSpec((OUT_DIM, OUT_DIM), lambda i: (0, 0)),
    ],
    out_specs=[
        pl.BlockSpec((_RB, NUM_RELS * OUT_DIM), lambda i: (i, 0)),
        pl.BlockSpec((_RB, 2 * OUT_DIM), lambda i: (i, 0)),
    ],
    out_shape=[
        jax.ShapeDtypeStruct((N_NODES, NUM_RELS * OUT_DIM), jnp.float32),
        jax.ShapeDtypeStruct((N_NODES, 2 * OUT_DIM), jnp.float32),
    ],
)


def _assemble_body(fc_ref, a0_ref, a1_ref, out_ref):
    out_ref[:, :OUT_DIM] = fc_ref[:, :OUT_DIM]
    out_ref[:, OUT_DIM:] = fc_ref[:, OUT_DIM:] + a0_ref[...] + a1_ref[...]


_AB = 2000
_ANB = N_NODES // _AB

_assemble = pl.pallas_call(
    _assemble_body,
    grid=(_ANB,),
    in_specs=[
        pl.BlockSpec((_AB, 2 * OUT_DIM), lambda i: (i, 0)),
        pl.BlockSpec((_AB, OUT_DIM), lambda i: (i, 0)),
        pl.BlockSpec((_AB, OUT_DIM), lambda i: (i + _ANB, 0)),
    ],
    out_specs=pl.BlockSpec((_AB, 2 * OUT_DIM), lambda i: (i, 0)),
    out_shape=jax.ShapeDtypeStruct((N_NODES, 2 * OUT_DIM), jnp.float32),
)


def kernel(feat, idx, edge_index, edge_type, embed, transform, weight,
           w_comp, self_loop_weight):
    src = edge_index[0]
    dst = edge_index[1]

    embed_gather, edge_agg = _sc_kernels()

    g = embed_gather(embed, idx.astype(jnp.int32))

    y, fc = _dense(feat, g, transform, weight, w_comp,
                   self_loop_weight)
    y_flat = y.reshape(NUM_RELS * N_NODES, OUT_DIM)

    gi = (src.astype(jnp.int32) * NUM_RELS
          + edge_type.astype(jnp.int32))
    dst_r = dst.astype(jnp.int32)
    zeros = jnp.zeros((N_NODES, OUT_DIM), jnp.float32)

    acc = edge_agg(y_flat, gi, dst_r, zeros)

    out = _assemble(fc, acc, acc)
    return out.reshape(-1, 2, OUT_DIM)
